# baseline (speedup 1.0000x reference)
import functools

import jax
import jax.numpy as jnp
from jax import lax
from jax.experimental import pallas as pl
from jax.experimental.pallas import tpu as pltpu
from jax.experimental.pallas import tpu_sc as plsc

B = 4096
F = 26
V = 100000
D = 16
H1 = 32
H2 = 32

NC = 2
NS = 16
NW = NC * NS
FH = F // 2
NG = B // 16

CH0 = 50048
CH1 = 49920
VTAIL = 128
B1D = 128
TOFF = B1D + CH1
BUF0 = CH0 + 16
BUF1 = TOFF + VTAIL
NU2 = 2 * FH
NU = NU2 + 2


def _sc_slab_fm(tview, first_tables, idx_flat, xvt, tail2, tail1):
    mesh = plsc.VectorSubcoreMesh(core_axis_name="c", subcore_axis_name="s")

    @functools.partial(
        pl.kernel,
        out_type=(
            jax.ShapeDtypeStruct((NW * B,), jnp.float32),
            jax.ShapeDtypeStruct((NW * B,), jnp.float32),
            jax.ShapeDtypeStruct((F * B,), jnp.float32),
        ),
        mesh=mesh,
        compiler_params=pltpu.CompilerParams(
            use_tc_tiling_on_sc=True, needs_layout_passes=False),
        scratch_types=[
            pltpu.VMEM((BUF0,), jnp.float32),
            pltpu.VMEM((BUF1,), jnp.float32),
            pltpu.VMEM((2 * B,), jnp.int32),
            pltpu.VMEM((2 * B,), jnp.float32),
            pltpu.VMEM((B,), jnp.float32),
            pltpu.VMEM((B,), jnp.float32),
            pltpu.VMEM((B,), jnp.float32),
            pltpu.SemaphoreType.DMA,
            pltpu.SemaphoreType.DMA,
            pltpu.SemaphoreType.DMA,
        ],
    )
    def body(tview_hbm, first_hbm, idx_hbm, xvt_hbm, tail2_hbm, tail1_hbm,
             s_hbm, sq_hbm, e1_hbm,
             slab0, slab1, idx_v, xv_v, s_acc, sq_acc, e1_acc,
             dsem0, dsem1, isem):
        wid = lax.axis_index("s") * NC + lax.axis_index("c")
        d = wid % D
        fh = wid // D
        fsafe = jnp.minimum(wid, F - 1)

        slabs = (slab0, slab1)
        dsems = (dsem0, dsem1)

        def field_of_slot(jf):
            return fh * FH + jf if jf < FH else fsafe

        def start_dma(u):
            t = u % 2
            if u < NU2:
                f = fh * FH + (u // 2)
                src_of = lambda off, sz: tview_hbm.at[f, d, pl.ds(off, sz)]
                tail_src = tail2_hbm.at[f, d, :]
            else:
                src_of = lambda off, sz: first_hbm.at[fsafe, pl.ds(off, sz)]
                tail_src = tail1_hbm.at[fsafe, :]
            if t == 0:
                return [pltpu.async_copy(
                    src_of(0, CH0), slabs[0].at[pl.ds(0, CH0)], dsems[0])]
            return [
                pltpu.async_copy(
                    src_of(CH0, CH1), slabs[1].at[pl.ds(B1D, CH1)], dsems[1]),
                pltpu.async_copy(
                    tail_src, slabs[1].at[pl.ds(TOFF, VTAIL)], dsems[1]),
            ]

        def start_idx_prefetch(jf):
            p = (jf % 2) * B
            f = field_of_slot(jf)
            c1 = pltpu.async_copy(
                idx_hbm.at[pl.ds(f * B, B)], idx_v.at[pl.ds(p, B)], isem)
            c2 = pltpu.async_copy(
                xvt_hbm.at[f, :], xv_v.at[pl.ds(p, B)], isem)
            return (c1, c2)

        zero16 = jnp.zeros((16,), jnp.float32)
        slab0[pl.ds(CH0, 16)] = zero16
        for z in range(B1D // 16):
            slab1[pl.ds(z * 16, 16)] = zero16

        icpy = start_idx_prefetch(0)
        dmas = {0: start_dma(0)}
        for c in icpy:
            c.wait()
        icpy = None

        for u in range(NU):
            jf, t = u // 2, u % 2
            if t == 0 and jf > 0:
                for c in icpy:
                    c.wait()
            if u + 1 < NU:
                dmas[u + 1] = start_dma(u + 1)
            for c in dmas.pop(u):
                c.wait()
            if t == 0 and jf + 1 <= FH:
                icpy = start_idx_prefetch(jf + 1)

            p = (jf % 2) * B
            buf = slabs[t]

            def g_body(g, _, _t=t, _u=u, _buf=buf, _p=p):
                sl = pl.ds(_p + g * 16, 16)
                asl = pl.ds(g * 16, 16)
                vi = idx_v[sl]
                if _t == 0:
                    vic = jnp.minimum(vi, CH0)
                else:
                    v2 = vi - (CH0 - B1D)
                    vic = jnp.maximum(v2, 0)
                    vict = v2 + (VTAIL - (V - CH0 - CH1))
                    vic = jnp.where(v2 >= TOFF, vict, vic)
                vals = plsc.load_gather(_buf, [vic])
                vs = vals * xv_v[sl]
                if _u == 0:
                    s_acc[asl] = vs
                    sq_acc[asl] = vs * vs
                elif _u < NU2:
                    plsc.addupdate(s_acc.at[asl], vs)
                    plsc.addupdate(sq_acc.at[asl], vs * vs)
                elif _u == NU2:
                    e1_acc[asl] = vs
                else:
                    plsc.addupdate(e1_acc.at[asl], vs)
                return 0

            lax.fori_loop(0, NG, g_body, 0, unroll=16)

        obase = wid * B
        pltpu.sync_copy(s_acc, s_hbm.at[pl.ds(obase, B)])
        pltpu.sync_copy(sq_acc, sq_hbm.at[pl.ds(obase, B)])
        pltpu.sync_copy(e1_acc, e1_hbm.at[pl.ds(fsafe * B, B)])

    return body(tview, first_tables, idx_flat, xvt, tail2, tail1)


def _tc_combine(s2, sq2, e1, W1t, b1, W2t, b2, bias):
    def tc_body(s_ref, sq_ref, e1_ref, W1t_ref, b1_ref, W2t_ref, b2_ref,
                bias_ref, out_ref):
        s = s_ref[0] + s_ref[1]
        sq = sq_ref[0] + sq_ref[1]
        fm2t = 0.5 * (s * s - sq)
        h1 = jnp.maximum(
            jnp.dot(W1t_ref[:], fm2t, preferred_element_type=jnp.float32)
            + b1_ref[:], 0.0)
        h2 = jnp.maximum(
            jnp.dot(W2t_ref[:], h1, preferred_element_type=jnp.float32)
            + b2_ref[:], 0.0)
        tot = (jnp.sum(fm2t, axis=0, keepdims=True)
               + jnp.sum(e1_ref[:], axis=0, keepdims=True)
               + jnp.sum(h2, axis=0, keepdims=True)
               + bias_ref[0, 0])
        out_ref[:] = tot

    return pl.pallas_call(
        tc_body,
        out_shape=jax.ShapeDtypeStruct((1, B), jnp.float32),
    )(s2, sq2, e1, W1t, b1, W2t, b2, bias)


def kernel(Xi, Xv, first_tables, second_tables, W1, b1, W2, b2, bias):
    tview = jnp.transpose(second_tables, (0, 2, 1))
    idx_flat = jnp.transpose(Xi, (1, 2, 0)).reshape(F * B)
    xvt = jnp.transpose(Xv)
    tail2 = tview[:, :, V - VTAIL:]
    tail1 = first_tables[:, V - VTAIL:]

    s_flat, sq_flat, e1_flat = _sc_slab_fm(
        tview, first_tables, idx_flat, xvt, tail2, tail1)
    s2 = s_flat.reshape(2, D, B)
    sq2 = sq_flat.reshape(2, D, B)
    e1 = e1_flat.reshape(F, B)

    out = _tc_combine(s2, sq2, e1, W1.T, b1.reshape(H1, 1), W2.T,
                      b2.reshape(H2, 1), bias.reshape(1, 1))
    return out[0]


# speedup vs baseline: 1.0536x; 1.0536x over previous
import functools

import jax
import jax.numpy as jnp
from jax import lax
from jax.experimental import pallas as pl
from jax.experimental.pallas import tpu as pltpu
from jax.experimental.pallas import tpu_sc as plsc

B = 4096
F = 26
V = 100000
D = 16
H1 = 32
H2 = 32

NC = 2
NS = 16
NW = NC * NS
FH = F // 2
NG = B // 16

CH0 = 50048
CH1 = 49920
VTAIL = 128
B1D = 128
TOFF = B1D + CH1
BUF0 = CH0 + 16
BUF1 = TOFF + VTAIL
NU2 = 2 * FH
NU = NU2 + 2


def _sc_slab_fm(tview, first_tables, idx_flat, xvt, tail2, tail1):
    mesh = plsc.VectorSubcoreMesh(core_axis_name="c", subcore_axis_name="s")

    @functools.partial(
        pl.kernel,
        out_type=(
            jax.ShapeDtypeStruct((NW * B,), jnp.float32),
            jax.ShapeDtypeStruct((NW * B,), jnp.float32),
            jax.ShapeDtypeStruct((F * B,), jnp.float32),
        ),
        mesh=mesh,
        compiler_params=pltpu.CompilerParams(
            use_tc_tiling_on_sc=True, needs_layout_passes=False),
        scratch_types=[
            pltpu.VMEM((BUF0,), jnp.float32),
            pltpu.VMEM((BUF1,), jnp.float32),
            pltpu.VMEM((2 * B,), jnp.int32),
            pltpu.VMEM((2 * B,), jnp.float32),
            pltpu.VMEM((B,), jnp.float32),
            pltpu.VMEM((B,), jnp.float32),
            pltpu.VMEM((B,), jnp.float32),
            pltpu.SemaphoreType.DMA,
            pltpu.SemaphoreType.DMA,
            pltpu.SemaphoreType.DMA,
        ],
    )
    def body(tview_hbm, first_hbm, idx_hbm, xvt_hbm, tail2_hbm, tail1_hbm,
             s_hbm, sq_hbm, e1_hbm,
             slab0, slab1, idx_v, xv_v, s_acc, sq_acc, e1_acc,
             dsem0, dsem1, isem):
        wid = lax.axis_index("s") * NC + lax.axis_index("c")
        d = wid % D
        fh = wid // D
        fsafe = jnp.minimum(wid, F - 1)

        slabs = (slab0, slab1)
        dsems = (dsem0, dsem1)

        def field_of_slot(jf):
            return fh * FH + jf if jf < FH else fsafe

        def start_dma(u):
            t = u % 2
            if u < NU2:
                f = fh * FH + (u // 2)
                src_of = lambda off, sz: tview_hbm.at[f, d, pl.ds(off, sz)]
                tail_src = tail2_hbm.at[f, d, :]
            else:
                src_of = lambda off, sz: first_hbm.at[fsafe, pl.ds(off, sz)]
                tail_src = tail1_hbm.at[fsafe, :]
            if t == 0:
                return [pltpu.async_copy(
                    src_of(0, CH0), slabs[0].at[pl.ds(0, CH0)], dsems[0])]
            return [
                pltpu.async_copy(
                    src_of(CH0, CH1), slabs[1].at[pl.ds(B1D, CH1)], dsems[1]),
                pltpu.async_copy(
                    tail_src, slabs[1].at[pl.ds(TOFF, VTAIL)], dsems[1]),
            ]

        def start_idx_prefetch(jf):
            p = (jf % 2) * B
            f = field_of_slot(jf)
            c1 = pltpu.async_copy(
                idx_hbm.at[pl.ds(f * B, B)], idx_v.at[pl.ds(p, B)], isem)
            c2 = pltpu.async_copy(
                xvt_hbm.at[f, :], xv_v.at[pl.ds(p, B)], isem)
            return (c1, c2)

        zero16 = jnp.zeros((16,), jnp.float32)
        slab0[pl.ds(CH0, 16)] = zero16
        for z in range(B1D // 16):
            slab1[pl.ds(z * 16, 16)] = zero16

        icpy = start_idx_prefetch(0)
        dmas = {0: start_dma(0)}
        for c in icpy:
            c.wait()
        icpy = None

        for u in range(NU):
            jf, t = u // 2, u % 2
            if t == 0 and jf > 0:
                for c in icpy:
                    c.wait()
            if u + 1 < NU:
                dmas[u + 1] = start_dma(u + 1)
            for c in dmas.pop(u):
                c.wait()
            if t == 0 and jf + 1 <= FH:
                icpy = start_idx_prefetch(jf + 1)

            p = (jf % 2) * B
            buf = slabs[t]

            def g_body(g, _, _t=t, _u=u, _buf=buf, _p=p):
                sl = pl.ds(_p + g * 16, 16)
                asl = pl.ds(g * 16, 16)
                vi = idx_v[sl]
                if _t == 0:
                    vic = jnp.minimum(vi, CH0)
                else:
                    v2 = vi - (CH0 - B1D)
                    vic = jnp.maximum(v2, 0)
                    vict = v2 + (VTAIL - (V - CH0 - CH1))
                    vic = jnp.where(v2 >= TOFF, vict, vic)
                vals = plsc.load_gather(_buf, [vic])
                vs = vals * xv_v[sl]
                if _u == 0:
                    s_acc[asl] = vs
                    sq_acc[asl] = vs * vs
                elif _u < NU2:
                    plsc.addupdate(s_acc.at[asl], vs)
                    plsc.addupdate(sq_acc.at[asl], vs * vs)
                elif _u == NU2:
                    e1_acc[asl] = vs
                else:
                    plsc.addupdate(e1_acc.at[asl], vs)
                return 0

            lax.fori_loop(0, NG, g_body, 0, unroll=4)

        obase = wid * B
        pltpu.sync_copy(s_acc, s_hbm.at[pl.ds(obase, B)])
        pltpu.sync_copy(sq_acc, sq_hbm.at[pl.ds(obase, B)])
        pltpu.sync_copy(e1_acc, e1_hbm.at[pl.ds(fsafe * B, B)])

    return body(tview, first_tables, idx_flat, xvt, tail2, tail1)


def _tc_combine(s2, sq2, e1, W1t, b1, W2t, b2, bias):
    def tc_body(s_ref, sq_ref, e1_ref, W1t_ref, b1_ref, W2t_ref, b2_ref,
                bias_ref, out_ref):
        s = s_ref[0] + s_ref[1]
        sq = sq_ref[0] + sq_ref[1]
        fm2t = 0.5 * (s * s - sq)
        h1 = jnp.maximum(
            jnp.dot(W1t_ref[:], fm2t, preferred_element_type=jnp.float32)
            + b1_ref[:], 0.0)
        h2 = jnp.maximum(
            jnp.dot(W2t_ref[:], h1, preferred_element_type=jnp.float32)
            + b2_ref[:], 0.0)
        tot = (jnp.sum(fm2t, axis=0, keepdims=True)
               + jnp.sum(e1_ref[:], axis=0, keepdims=True)
               + jnp.sum(h2, axis=0, keepdims=True)
               + bias_ref[0, 0])
        out_ref[:] = tot

    return pl.pallas_call(
        tc_body,
        out_shape=jax.ShapeDtypeStruct((1, B), jnp.float32),
    )(s2, sq2, e1, W1t, b1, W2t, b2, bias)


def kernel(Xi, Xv, first_tables, second_tables, W1, b1, W2, b2, bias):
    tview = jnp.transpose(second_tables, (0, 2, 1))
    idx_flat = jnp.transpose(Xi, (1, 2, 0)).reshape(F * B)
    xvt = jnp.transpose(Xv)
    tail2 = tview[:, :, V - VTAIL:]
    tail1 = first_tables[:, V - VTAIL:]

    s_flat, sq_flat, e1_flat = _sc_slab_fm(
        tview, first_tables, idx_flat, xvt, tail2, tail1)
    s2 = s_flat.reshape(2, D, B)
    sq2 = sq_flat.reshape(2, D, B)
    e1 = e1_flat.reshape(F, B)

    out = _tc_combine(s2, sq2, e1, W1.T, b1.reshape(H1, 1), W2.T,
                      b2.reshape(H2, 1), bias.reshape(1, 1))
    return out[0]
